# Initial kernel scaffold; baseline (speedup 1.0000x reference)
import jax
import jax.numpy as jnp
from jax.experimental import pallas as pl


def kernel(hidden_states, topk_weights, topk_ids, w_up, w_down):
    raise NotImplementedError("write your pallas kernel here")


# speedup vs baseline: 1.6955x; 1.6955x over previous
import functools

import jax
import jax.numpy as jnp
from jax import lax
from jax.experimental import pallas as pl
from jax.experimental.pallas import tpu as pltpu
from jax.experimental.pallas import tpu_sc as plsc

BM = 128
NJ = 4
GW = 32
CW = 16


def _routing(topk_ids, topk_weights, N, K, E, NB, P):
    NK = N * K
    ids = topk_ids.reshape(NK).astype(jnp.int32)
    onehot = (ids[:, None] == jnp.arange(E, dtype=jnp.int32)[None, :]).astype(
        jnp.int32)
    csum = jnp.cumsum(onehot, axis=0)
    counts = csum[-1]
    rank = jnp.take_along_axis(csum, ids[:, None], 1)[:, 0] - 1
    blocks_e = (counts + BM - 1) // BM
    bends = jnp.cumsum(blocks_e)
    bstart = bends - blocks_e
    block_expert = jnp.minimum(
        jnp.searchsorted(bends, jnp.arange(NB, dtype=jnp.int32), side="right"),
        E - 1).astype(jnp.int32)
    pp = (bstart[ids] * BM + rank).astype(jnp.int32)
    tok = jnp.arange(NK, dtype=jnp.int32) // K
    tok_padded = jnp.zeros((P,), jnp.int32).at[pp].set(tok)
    tw_padded = jnp.zeros((P,), jnp.float32).at[pp].set(
        topk_weights.reshape(NK).astype(jnp.float32))
    ppN = pp.reshape(N, K)
    return block_expert, tok_padded, tw_padded, ppN


def _sc_dispatch(hidden_states, tok_padded, P, D):
    info = plsc.get_sparse_core_info()
    NC, NS = info.num_cores, info.num_subcores
    NW = NC * NS
    per_w = P // NW
    mesh = plsc.VectorSubcoreMesh(core_axis_name="c", subcore_axis_name="s")

    @functools.partial(
        pl.kernel, mesh=mesh,
        out_type=jax.ShapeDtypeStruct((P, D), jnp.float32),
        scratch_types=[pltpu.VMEM((GW,), jnp.int32),
                       pltpu.VMEM((GW, D), jnp.float32),
                       pltpu.SemaphoreType.DMA])
    def gather_kernel(hs_hbm, tok_hbm, gx_hbm, idx_v, rows_v, sem):
        wid = lax.axis_index("s") * NC + lax.axis_index("c")
        base = wid * per_w

        @pl.loop(0, per_w, step=GW)
        def _(c):
            pltpu.sync_copy(tok_hbm.at[pl.ds(base + c, GW)], idx_v)
            pltpu.async_copy(hs_hbm.at[idx_v], rows_v, sem).wait()
            pltpu.sync_copy(rows_v, gx_hbm.at[pl.ds(base + c, GW)])

    return gather_kernel(hidden_states, tok_padded)


def _tc_grouped_mlp(block_expert, gx, w_up, w_down, tw_padded,
                    N, D, H, E, NB, P, interpret=False):
    HT = H // NJ
    tw2d = tw_padded.reshape(P, 1)

    def body(eids_ref, wg_ref, wl_ref, wd_ref, gx_ref, tw_ref, o_ref):
        j = pl.program_id(0)
        b = pl.program_id(1)
        xb = gx_ref[...].astype(jnp.bfloat16)
        wg = wg_ref[0].astype(jnp.bfloat16)
        wl = wl_ref[0].astype(jnp.bfloat16)
        wd = wd_ref[0].astype(jnp.bfloat16)
        hg = lax.dot_general(xb, wg, (((1,), (1,)), ((), ())),
                             preferred_element_type=jnp.float32)
        hl = lax.dot_general(xb, wl, (((1,), (1,)), ((), ())),
                             preferred_element_type=jnp.float32)
        g = (hg * jax.nn.sigmoid(hg)) * hl
        part = lax.dot_general(g.astype(jnp.bfloat16), wd,
                               (((1,), (1,)), ((), ())),
                               preferred_element_type=jnp.float32)
        rows = pl.ds(b * BM, BM)

        @pl.when(j == 0)
        def _():
            o_ref[rows, :] = part

        @pl.when(j > 0)
        def _():
            o_ref[rows, :] = o_ref[rows, :] + part

        @pl.when(j == NJ - 1)
        def _():
            o_ref[rows, :] = o_ref[rows, :] * tw_ref[...]

    grid_spec = pltpu.PrefetchScalarGridSpec(
        num_scalar_prefetch=1,
        grid=(NJ, NB),
        in_specs=[
            pl.BlockSpec((1, HT, D), lambda j, b, eids: (eids[b], j, 0)),
            pl.BlockSpec((1, HT, D), lambda j, b, eids: (eids[b], NJ + j, 0)),
            pl.BlockSpec((1, D, HT), lambda j, b, eids: (eids[b], 0, j)),
            pl.BlockSpec((BM, D), lambda j, b, eids: (b, 0)),
            pl.BlockSpec((BM, 1), lambda j, b, eids: (b, 0)),
        ],
        out_specs=pl.BlockSpec((P, D), lambda j, b, eids: (0, 0)),
    )
    return pl.pallas_call(
        body,
        grid_spec=grid_spec,
        out_shape=jax.ShapeDtypeStruct((P, D), jnp.float32),
        compiler_params=pltpu.CompilerParams(
            dimension_semantics=("arbitrary", "arbitrary"),
            vmem_limit_bytes=100 * 1024 * 1024,
        ),
        interpret=interpret,
    )(block_expert, w_up, w_up, w_down, gx, tw2d)


def _sc_combine(y, ppN, N, K, D):
    info = plsc.get_sparse_core_info()
    NC, NS = info.num_cores, info.num_subcores
    NW = NC * NS
    per_w = N // NW
    mesh = plsc.VectorSubcoreMesh(core_axis_name="c", subcore_axis_name="s")
    idx0 = ppN[:, 0]
    idx1 = ppN[:, 1]

    @functools.partial(
        pl.kernel, mesh=mesh,
        out_type=jax.ShapeDtypeStruct((N, D), jnp.float32),
        scratch_types=[pltpu.VMEM((CW,), jnp.int32),
                       pltpu.VMEM((CW,), jnp.int32),
                       pltpu.VMEM((CW, D), jnp.float32),
                       pltpu.VMEM((CW, D), jnp.float32),
                       pltpu.SemaphoreType.DMA])
    def combine_kernel(y_hbm, i0_hbm, i1_hbm, out_hbm,
                       i0_v, i1_v, acc_v, rows_v, sem):
        wid = lax.axis_index("s") * NC + lax.axis_index("c")
        base = wid * per_w

        @pl.loop(0, per_w, step=CW)
        def _(c):
            off = base + c
            pltpu.sync_copy(i0_hbm.at[pl.ds(off, CW)], i0_v)
            pltpu.sync_copy(i1_hbm.at[pl.ds(off, CW)], i1_v)
            pltpu.async_copy(y_hbm.at[i0_v], acc_v, sem).wait()
            pltpu.async_copy(y_hbm.at[i1_v], rows_v, sem).wait()

            @pl.loop(0, CW)
            def _(t):
                @pl.loop(0, D, step=16)
                def _(d):
                    acc_v[t, pl.ds(d, 16)] += rows_v[t, pl.ds(d, 16)]

            pltpu.sync_copy(acc_v, out_hbm.at[pl.ds(off, CW)])

    return combine_kernel(y, idx0, idx1)


def kernel(hidden_states, topk_weights, topk_ids, w_up, w_down):
    N, D = hidden_states.shape
    K = topk_ids.shape[1]
    E = w_up.shape[0]
    H = w_down.shape[2]
    NB = (N * K) // BM + E
    P = NB * BM

    block_expert, tok_padded, tw_padded, ppN = _routing(
        topk_ids, topk_weights, N, K, E, NB, P)
    gx = _sc_dispatch(hidden_states, tok_padded, P, D)
    y = _tc_grouped_mlp(block_expert, gx, w_up, w_down, tw_padded,
                        N, D, H, E, NB, P)
    return _sc_combine(y, ppN, N, K, D)
